# baseline (speedup 1.0000x reference)
import functools

import jax
import jax.numpy as jnp
from jax import lax
from jax.experimental import pallas as pl
from jax.experimental.pallas import tpu as pltpu
from jax.experimental.pallas import tpu_sc as plsc

_N = 10000
_E = 320000
_EPAD = 327680
_IDXROWS = _EPAD // 128
_ROWS_PER_W = _IDXROWS // 32
_NCHUNK = 10
_GW = 48
_SW = 32
_NACC = _N + 16
_RPT = _NACC // 16


def _lrelu(x):
    return jnp.where(x > 0, x, 0.01 * x)


_mesh = plsc.VectorSubcoreMesh(core_axis_name="c", subcore_axis_name="s")
_sc_params = pltpu.CompilerParams(use_tc_tiling_on_sc=False)


_GCH = 1024


@functools.partial(
    pl.kernel,
    out_type=jax.ShapeDtypeStruct((_EPAD, _GW), jnp.float32),
    mesh=_mesh,
    compiler_params=_sc_params,
    scratch_types=[
        pltpu.VMEM((8, 128), jnp.int32),
        pltpu.VMEM((8, 128), jnp.int32),
        pltpu.VMEM((8, 128), jnp.int32),
        pltpu.VMEM((8, 128), jnp.int32),
        pltpu.VMEM((_GCH, _GW), jnp.float32),
        pltpu.VMEM((_GCH, _GW), jnp.float32),
        pltpu.SemaphoreType.DMA,
        pltpu.SemaphoreType.DMA,
        pltpu.SemaphoreType.DMA,
        pltpu.SemaphoreType.DMA,
    ],
)
def _sc_gather(t1, t2, dst2d, src2d, out,
               ida, idb, isa, isb, ba, bb, sga, sgb, swa, swb):
    c = lax.axis_index("c")
    s = lax.axis_index("s")
    wid = s * 2 + c
    base = wid * _ROWS_PER_W
    idd = [ida, idb]
    ids = [isa, isb]
    buf = [ba, bb]
    sg = [sga, sgb]
    sw = [swa, swb]

    def load_idx(ci, k):
        r0 = base + ci * 8
        pltpu.sync_copy(dst2d.at[pl.ds(r0, 8)], idd[k])
        pltpu.sync_copy(src2d.at[pl.ds(r0, 8)], ids[k])

    def fire_t1(k):
        return [pltpu.async_copy(t1.at[idd[k].at[j]],
                                 buf[k].at[pl.ds(j * 128, 128)], sg[k])
                for j in range(8)]

    def fire_t2(k):
        return [pltpu.async_copy(t2.at[ids[k].at[j]],
                                 buf[k].at[pl.ds(j * 128, 128)], sg[k],
                                 add=True)
                for j in range(8)]

    wo = [None, None]
    t1g = [None, None]
    load_idx(0, 0)
    t1g[0] = fire_t1(0)
    for ci in range(_NCHUNK):
        k = ci & 1
        kn = 1 - k
        if ci + 1 < _NCHUNK:
            load_idx(ci + 1, kn)
            if wo[kn] is not None:
                wo[kn].wait()
            t1g[kn] = fire_t1(kn)
        for cp in t1g[k]:
            cp.wait()
        for cp in fire_t2(k):
            cp.wait()
        wo[k] = pltpu.async_copy(
            buf[k], out.at[pl.ds((base + ci * 8) * 128, _GCH)], sw[k])
    wo[0].wait()
    wo[1].wait()


@functools.partial(
    pl.kernel,
    out_type=jax.ShapeDtypeStruct((2, _NACC, _SW), jnp.float32),
    mesh=_mesh,
    compiler_params=_sc_params,
    scratch_types=[
        pltpu.VMEM((8, 128), jnp.int32),
        pltpu.VMEM((1024, _SW), jnp.float32),
        pltpu.VMEM_SHARED((_NACC, _SW), jnp.float32),
    ],
)
def _sc_scatter(r_hbm, dst2d, zrows, out, idxv, rb, acc):
    c = lax.axis_index("c")
    s = lax.axis_index("s")
    wid = s * 2 + c
    base = wid * _ROWS_PER_W

    pltpu.sync_copy(zrows, rb.at[pl.ds(0, _RPT)])
    pltpu.sync_copy(rb.at[pl.ds(0, _RPT)], acc.at[pl.ds(s * _RPT, _RPT)])
    plsc.subcore_barrier()

    def chunk(ci, carry):
        r0 = base + ci * 8
        pltpu.sync_copy(dst2d.at[pl.ds(r0, 8)], idxv)
        pltpu.sync_copy(r_hbm.at[pl.ds(r0 * 128, 1024)], rb)
        for j in range(8):
            pltpu.sync_copy(rb.at[pl.ds(j * 128, 128)],
                            acc.at[idxv.at[j]], add=True)
        return carry

    lax.fori_loop(0, _NCHUNK, chunk, 0)
    plsc.subcore_barrier()
    pltpu.sync_copy(acc.at[pl.ds(s * _RPT, _RPT)], rb.at[pl.ds(0, _RPT)])
    pltpu.sync_copy(rb.at[pl.ds(0, _RPT)], out.at[c, pl.ds(s * _RPT, _RPT)])


def _readin_body(h_ref, w_ref, b_ref, o_ref):
    o_ref[...] = _lrelu(
        jnp.dot(h_ref[...], w_ref[...], preferred_element_type=jnp.float32)
        + b_ref[...])


def _prep_body(x_ref, p_ref, g_ref, bt_ref, w1a_ref, w1b_ref, b1_ref,
               y_ref, t1_ref, t2_ref):
    x = x_ref[...]
    mu = jnp.mean(x, axis=0, keepdims=True)
    xc = x - mu
    var = jnp.mean(xc * xc, axis=0, keepdims=True)
    y = _lrelu(xc / jnp.sqrt(var + 1e-5) * g_ref[...] + bt_ref[...])
    y_ref[...] = y
    p = p_ref[...]
    zpad = jnp.zeros((_N, _GW - 35), jnp.float32)
    a = jnp.dot(y, w1a_ref[...], preferred_element_type=jnp.float32) + b1_ref[...]
    b = jnp.dot(y, w1b_ref[...], preferred_element_type=jnp.float32)
    t1_ref[...] = jnp.concatenate([a, p, zpad], axis=1)
    t2_ref[...] = jnp.concatenate([b, -p, zpad], axis=1)


_K = 8
_GP = _K * _GW
_MP = _K * 16
_UP = _K * 32
_XP = _K * 4
_RP = _K * _SW


def _edge_packed_weights(w1c, W2, b2, cW1, cb1, cW2, cb2):
    import numpy as np
    eyeK = np.eye(_K, dtype=np.float32)
    S1 = np.zeros((_GW, _GW), np.float32)
    S1[32:35, 0:32] = 1.0
    D1 = np.zeros((_GW, 4), np.float32)
    D1[32:35, 0:3] = np.eye(3)
    Pm1 = np.zeros((16, _SW), np.float32)
    Pm1[:, :16] = np.eye(16)
    Px1 = np.zeros((4, _SW), np.float32)
    Px1[0:3, 16:19] = np.eye(3)
    onesK = np.ones((_K,), np.float32)

    w2pad = jnp.pad(W2, ((0, _GW - 32), (0, 0)))
    cw2pad = jnp.pad(cW2, ((0, 0), (0, 1)))
    return dict(
        sp=jnp.asarray(np.kron(eyeK, S1)),
        dsel=jnp.asarray(np.kron(eyeK, D1)),
        pm=jnp.asarray(np.kron(eyeK, Pm1)),
        px=jnp.asarray(np.kron(eyeK, Px1)),
        w1cp=jnp.kron(onesK, jnp.pad(w1c.reshape(-1), (0, _GW - 32))).reshape(1, _GP),
        w2p=jnp.kron(jnp.asarray(eyeK), w2pad),
        b2p=jnp.kron(onesK, b2.reshape(-1)).reshape(1, _MP),
        cw1p=jnp.kron(jnp.asarray(eyeK), cW1),
        cb1p=jnp.kron(onesK, cb1.reshape(-1)).reshape(1, _UP),
        cw2p=jnp.kron(jnp.asarray(eyeK), cw2pad),
        cb2p=jnp.kron(onesK, jnp.pad(cb2.reshape(-1), (0, 1))).reshape(1, _XP),
    )


def _edge_body(g_ref, sp_ref, dsel_ref, pm_ref, px_ref, w1cp_ref, w2p_ref,
               b2p_ref, cw1p_ref, cb1p_ref, cw2p_ref, cb2p_ref, o_ref):
    g = g_ref[...]
    sq = g * g
    dist = jnp.sqrt(
        jnp.dot(sq, sp_ref[...], preferred_element_type=jnp.float32) + 1e-8)
    t = _lrelu(g + dist * w1cp_ref[...])
    m = jnp.dot(t, w2p_ref[...], preferred_element_type=jnp.float32) + b2p_ref[...]
    u = _lrelu(jnp.dot(m, cw1p_ref[...], preferred_element_type=jnp.float32)
               + cb1p_ref[...])
    cw = (jnp.dot(u, cw2p_ref[...], preferred_element_type=jnp.float32)
          + cb2p_ref[...])
    dsv = jnp.dot(g, dsel_ref[...], preferred_element_type=jnp.float32)
    xij = dsv * cw
    o_ref[...] = (jnp.dot(m, pm_ref[...], preferred_element_type=jnp.float32)
                  + jnp.dot(xij, px_ref[...], preferred_element_type=jnp.float32))


def _node_body(x_ref, p_ref, y_ref, parts_ref, nw1_ref, nb1_ref, nw2_ref,
               nb2_ref, xo_ref, po_ref):
    agg = parts_ref[0, :_N, :] + parts_ref[1, :_N, :]
    aggm = agg[:, :16]
    aggx = agg[:, 16:19]
    cat = jnp.concatenate([y_ref[...], aggm], axis=1)
    t = _lrelu(jnp.dot(cat, nw1_ref[...], preferred_element_type=jnp.float32)
               + nb1_ref[...])
    hn = jnp.dot(t, nw2_ref[...], preferred_element_type=jnp.float32) + nb2_ref[...]
    xo_ref[...] = x_ref[...] + hn
    po_ref[...] = p_ref[...] + aggx


def _readout_body(x_ref, w_ref, b_ref, o_ref):
    o_ref[...] = (jnp.dot(x_ref[...], w_ref[...],
                          preferred_element_type=jnp.float32) + b_ref[...])


_RB = 256


def _edge_call(g, w1c, w2, b2, cw1, cb1, cw2, cb2):
    wd = _edge_packed_weights(w1c, w2, b2, cw1, cb1, cw2, cb2)
    gp = g.reshape(_EPAD // _K, _GP)
    nblk = gp.shape[0] // _RB
    full = lambda a: pl.BlockSpec(a.shape, lambda i: tuple(0 for _ in a.shape))
    args = [wd["sp"], wd["dsel"], wd["pm"], wd["px"], wd["w1cp"], wd["w2p"],
            wd["b2p"], wd["cw1p"], wd["cb1p"], wd["cw2p"], wd["cb2p"]]
    rp = pl.pallas_call(
        _edge_body,
        grid=(nblk,),
        in_specs=[pl.BlockSpec((_RB, _GP), lambda i: (i, 0))] +
                 [full(a) for a in args],
        out_specs=pl.BlockSpec((_RB, _RP), lambda i: (i, 0)),
        out_shape=jax.ShapeDtypeStruct((_EPAD // _K, _RP), jnp.float32),
    )(gp, *args)
    return rp.reshape(_EPAD, _SW)


def kernel(h, pos, edge_index, readin_W, readin_b, bn_gamma, bn_beta,
           edge_W1, edge_b1, edge_W2, edge_b2,
           coord_W1, coord_b1, coord_W2, coord_b2,
           node_W1, node_b1, node_W2, node_b2,
           readout_W, readout_b):
    src = edge_index[0]
    dst = edge_index[1]
    npad = _EPAD - _E
    zi = jnp.zeros((npad,), jnp.int32)
    dst_g = jnp.concatenate([dst, zi]).reshape(_IDXROWS, 128)
    src_g = jnp.concatenate([src, zi]).reshape(_IDXROWS, 128)
    dst_s = jnp.concatenate([dst, jnp.full((npad,), _N, jnp.int32)]
                            ).reshape(_IDXROWS, 128)
    zrows = jnp.zeros((_RPT, _SW), jnp.float32)

    x = pl.pallas_call(
        _readin_body,
        out_shape=jax.ShapeDtypeStruct((_N, 32), jnp.float32),
    )(h, readin_W, readin_b.reshape(1, -1))

    p = pos
    L = bn_gamma.shape[0]
    for l in range(L):
        w1a = edge_W1[l, :32]
        w1b = edge_W1[l, 32:64]
        w1c = edge_W1[l, 64:65]
        y, t1, t2 = pl.pallas_call(
            _prep_body,
            out_shape=[
                jax.ShapeDtypeStruct((_N, 32), jnp.float32),
                jax.ShapeDtypeStruct((_N, _GW), jnp.float32),
                jax.ShapeDtypeStruct((_N, _GW), jnp.float32),
            ],
        )(x, p, bn_gamma[l].reshape(1, -1), bn_beta[l].reshape(1, -1),
          w1a, w1b, edge_b1[l].reshape(1, -1))

        g = _sc_gather(t1, t2, dst_g, src_g)

        r = _edge_call(g, w1c, edge_W2[l], edge_b2[l].reshape(1, -1),
                       coord_W1[l], coord_b1[l].reshape(1, -1),
                       coord_W2[l], coord_b2[l].reshape(1, -1))

        parts = _sc_scatter(r, dst_s, zrows)

        x, p = pl.pallas_call(
            _node_body,
            out_shape=[
                jax.ShapeDtypeStruct((_N, 32), jnp.float32),
                jax.ShapeDtypeStruct((_N, 3), jnp.float32),
            ],
        )(x, p, y, parts, node_W1[l], node_b1[l].reshape(1, -1),
          node_W2[l], node_b2[l].reshape(1, -1))

    return pl.pallas_call(
        _readout_body,
        out_shape=jax.ShapeDtypeStruct((_N, 128), jnp.float32),
    )(x, readout_W, readout_b.reshape(1, -1))


# speedup vs baseline: 1.0979x; 1.0979x over previous
import functools

import jax
import jax.numpy as jnp
from jax import lax
from jax.experimental import pallas as pl
from jax.experimental.pallas import tpu as pltpu
from jax.experimental.pallas import tpu_sc as plsc

_N = 10000
_E = 320000
_EPAD = 327680
_IDXROWS = _EPAD // 128
_ROWS_PER_W = _IDXROWS // 32
_NCHUNK = 10
_GW = 48
_SW = 32
_NACC = _N + 16
_RPT = _NACC // 16


def _lrelu(x):
    return jnp.where(x > 0, x, 0.01 * x)


_mesh = plsc.VectorSubcoreMesh(core_axis_name="c", subcore_axis_name="s")
_sc_params = pltpu.CompilerParams(use_tc_tiling_on_sc=False)


_GCH = 1024


@functools.partial(
    pl.kernel,
    out_type=jax.ShapeDtypeStruct((_EPAD, _GW), jnp.float32),
    mesh=_mesh,
    compiler_params=_sc_params,
    scratch_types=[
        pltpu.VMEM((8, 128), jnp.int32),
        pltpu.VMEM((8, 128), jnp.int32),
        pltpu.VMEM((8, 128), jnp.int32),
        pltpu.VMEM((8, 128), jnp.int32),
        pltpu.VMEM((_GCH, _GW), jnp.float32),
        pltpu.VMEM((_GCH, _GW), jnp.float32),
        pltpu.SemaphoreType.DMA,
        pltpu.SemaphoreType.DMA,
        pltpu.SemaphoreType.DMA,
        pltpu.SemaphoreType.DMA,
    ],
)
def _sc_gather(t1, t2, dst2d, src2d, out,
               ida, idb, isa, isb, ba, bb, sga, sgb, swa, swb):
    c = lax.axis_index("c")
    s = lax.axis_index("s")
    wid = s * 2 + c
    base = wid * _ROWS_PER_W
    idd = [ida, idb]
    ids = [isa, isb]
    buf = [ba, bb]
    sg = [sga, sgb]
    sw = [swa, swb]

    def load_idx(ci, k):
        r0 = base + ci * 8
        pltpu.sync_copy(dst2d.at[pl.ds(r0, 8)], idd[k])
        pltpu.sync_copy(src2d.at[pl.ds(r0, 8)], ids[k])

    def fire_t1(k):
        return [pltpu.async_copy(t1.at[idd[k].at[j]],
                                 buf[k].at[pl.ds(j * 128, 128)], sg[k])
                for j in range(8)]

    def fire_t2(k):
        return [pltpu.async_copy(t2.at[ids[k].at[j]],
                                 buf[k].at[pl.ds(j * 128, 128)], sg[k],
                                 add=True)
                for j in range(8)]

    wo = [None, None]
    t1g = [None, None]
    load_idx(0, 0)
    t1g[0] = fire_t1(0)
    for ci in range(_NCHUNK):
        k = ci & 1
        kn = 1 - k
        if ci + 1 < _NCHUNK:
            load_idx(ci + 1, kn)
            if wo[kn] is not None:
                wo[kn].wait()
            t1g[kn] = fire_t1(kn)
        for cp in t1g[k]:
            cp.wait()
        for cp in fire_t2(k):
            cp.wait()
        wo[k] = pltpu.async_copy(
            buf[k], out.at[pl.ds((base + ci * 8) * 128, _GCH)], sw[k])
    wo[0].wait()
    wo[1].wait()


@functools.partial(
    pl.kernel,
    out_type=jax.ShapeDtypeStruct((2, _NACC, _SW), jnp.float32),
    mesh=_mesh,
    compiler_params=_sc_params,
    scratch_types=[
        pltpu.VMEM((8, 128), jnp.int32),
        pltpu.VMEM((1024, _SW), jnp.float32),
        pltpu.VMEM_SHARED((_NACC, _SW), jnp.float32),
    ],
)
def _sc_scatter(r_hbm, dst2d, zrows, out, idxv, rb, acc):
    c = lax.axis_index("c")
    s = lax.axis_index("s")
    wid = s * 2 + c
    base = wid * _ROWS_PER_W

    pltpu.sync_copy(zrows, rb.at[pl.ds(0, _RPT)])
    pltpu.sync_copy(rb.at[pl.ds(0, _RPT)], acc.at[pl.ds(s * _RPT, _RPT)])
    plsc.subcore_barrier()

    def chunk(ci, carry):
        r0 = base + ci * 8
        pltpu.sync_copy(dst2d.at[pl.ds(r0, 8)], idxv)
        pltpu.sync_copy(r_hbm.at[pl.ds(r0 * 128, 1024)], rb)
        for j in range(8):
            pltpu.sync_copy(rb.at[pl.ds(j * 128, 128)],
                            acc.at[idxv.at[j]], add=True)
        return carry

    lax.fori_loop(0, _NCHUNK, chunk, 0)
    plsc.subcore_barrier()
    pltpu.sync_copy(acc.at[pl.ds(s * _RPT, _RPT)], rb.at[pl.ds(0, _RPT)])
    pltpu.sync_copy(rb.at[pl.ds(0, _RPT)], out.at[c, pl.ds(s * _RPT, _RPT)])


def _readin_body(h_ref, w_ref, b_ref, o_ref):
    o_ref[...] = _lrelu(
        jnp.dot(h_ref[...], w_ref[...], preferred_element_type=jnp.float32)
        + b_ref[...])


def _prep_body(x_ref, p_ref, g_ref, bt_ref, w1a_ref, w1b_ref, b1_ref,
               y_ref, t1_ref, t2_ref):
    x = x_ref[...]
    mu = jnp.mean(x, axis=0, keepdims=True)
    xc = x - mu
    var = jnp.mean(xc * xc, axis=0, keepdims=True)
    y = _lrelu(xc / jnp.sqrt(var + 1e-5) * g_ref[...] + bt_ref[...])
    y_ref[...] = y
    p = p_ref[...]
    zpad = jnp.zeros((_N, _GW - 35), jnp.float32)
    a = jnp.dot(y, w1a_ref[...], preferred_element_type=jnp.float32) + b1_ref[...]
    b = jnp.dot(y, w1b_ref[...], preferred_element_type=jnp.float32)
    t1_ref[...] = jnp.concatenate([a, p, zpad], axis=1)
    t2_ref[...] = jnp.concatenate([b, -p, zpad], axis=1)


_K = 8
_GP = _K * _GW
_MP = _K * 16
_UP = _K * 32
_XP = _K * 4
_RP = _K * _SW


def _edge_packed_weights(w1c, W2, b2, cW1, cb1, cW2, cb2):
    import numpy as np
    eyeK = np.eye(_K, dtype=np.float32)
    S1 = np.zeros((_GW, _GW), np.float32)
    S1[32:35, 0:32] = 1.0
    D1 = np.zeros((_GW, 4), np.float32)
    D1[32:35, 0:3] = np.eye(3)
    Pm1 = np.zeros((16, _SW), np.float32)
    Pm1[:, :16] = np.eye(16)
    Px1 = np.zeros((4, _SW), np.float32)
    Px1[0:3, 16:19] = np.eye(3)
    onesK = np.ones((_K,), np.float32)

    w2pad = jnp.pad(W2, ((0, _GW - 32), (0, 0)))
    cw2pad = jnp.pad(cW2, ((0, 0), (0, 1)))
    return dict(
        sp=jnp.asarray(np.kron(eyeK, S1)),
        dsel=jnp.asarray(np.kron(eyeK, D1)),
        pm=jnp.asarray(np.kron(eyeK, Pm1)),
        px=jnp.asarray(np.kron(eyeK, Px1)),
        w1cp=jnp.kron(onesK, jnp.pad(w1c.reshape(-1), (0, _GW - 32))).reshape(1, _GP),
        w2p=jnp.kron(jnp.asarray(eyeK), w2pad),
        b2p=jnp.kron(onesK, b2.reshape(-1)).reshape(1, _MP),
        cw1p=jnp.kron(jnp.asarray(eyeK), cW1),
        cb1p=jnp.kron(onesK, cb1.reshape(-1)).reshape(1, _UP),
        cw2p=jnp.kron(jnp.asarray(eyeK), cw2pad),
        cb2p=jnp.kron(onesK, jnp.pad(cb2.reshape(-1), (0, 1))).reshape(1, _XP),
    )


def _edge_body(g_ref, sp_ref, dsel_ref, pm_ref, px_ref, w1cp_ref, w2p_ref,
               b2p_ref, cw1p_ref, cb1p_ref, cw2p_ref, cb2p_ref, o_ref):
    g = g_ref[...]
    sq = g * g
    dist = jnp.sqrt(
        jnp.dot(sq, sp_ref[...], preferred_element_type=jnp.float32) + 1e-8)
    t = _lrelu(g + dist * w1cp_ref[...])
    m = jnp.dot(t, w2p_ref[...], preferred_element_type=jnp.float32) + b2p_ref[...]
    u = _lrelu(jnp.dot(m, cw1p_ref[...], preferred_element_type=jnp.float32)
               + cb1p_ref[...])
    cw = (jnp.dot(u, cw2p_ref[...], preferred_element_type=jnp.float32)
          + cb2p_ref[...])
    dsv = jnp.dot(g, dsel_ref[...], preferred_element_type=jnp.float32)
    xij = dsv * cw
    o_ref[...] = (jnp.dot(m, pm_ref[...], preferred_element_type=jnp.float32)
                  + jnp.dot(xij, px_ref[...], preferred_element_type=jnp.float32))


def _node_body(x_ref, p_ref, y_ref, parts_ref, nw1_ref, nb1_ref, nw2_ref,
               nb2_ref, xo_ref, po_ref):
    agg = parts_ref[0, :_N, :] + parts_ref[1, :_N, :]
    aggm = agg[:, :16]
    aggx = agg[:, 16:19]
    cat = jnp.concatenate([y_ref[...], aggm], axis=1)
    t = _lrelu(jnp.dot(cat, nw1_ref[...], preferred_element_type=jnp.float32)
               + nb1_ref[...])
    hn = jnp.dot(t, nw2_ref[...], preferred_element_type=jnp.float32) + nb2_ref[...]
    xo_ref[...] = x_ref[...] + hn
    po_ref[...] = p_ref[...] + aggx


def _readout_body(x_ref, w_ref, b_ref, o_ref):
    o_ref[...] = (jnp.dot(x_ref[...], w_ref[...],
                          preferred_element_type=jnp.float32) + b_ref[...])


_RB = 512


def _edge_call(g, w1c, w2, b2, cw1, cb1, cw2, cb2):
    wd = _edge_packed_weights(w1c, w2, b2, cw1, cb1, cw2, cb2)
    gp = g.reshape(_EPAD // _K, _GP)
    nblk = gp.shape[0] // _RB
    full = lambda a: pl.BlockSpec(a.shape, lambda i: tuple(0 for _ in a.shape))
    args = [wd["sp"], wd["dsel"], wd["pm"], wd["px"], wd["w1cp"], wd["w2p"],
            wd["b2p"], wd["cw1p"], wd["cb1p"], wd["cw2p"], wd["cb2p"]]
    rp = pl.pallas_call(
        _edge_body,
        grid=(nblk,),
        in_specs=[pl.BlockSpec((_RB, _GP), lambda i: (i, 0))] +
                 [full(a) for a in args],
        out_specs=pl.BlockSpec((_RB, _RP), lambda i: (i, 0)),
        out_shape=jax.ShapeDtypeStruct((_EPAD // _K, _RP), jnp.float32),
    )(gp, *args)
    return rp.reshape(_EPAD, _SW)


def kernel(h, pos, edge_index, readin_W, readin_b, bn_gamma, bn_beta,
           edge_W1, edge_b1, edge_W2, edge_b2,
           coord_W1, coord_b1, coord_W2, coord_b2,
           node_W1, node_b1, node_W2, node_b2,
           readout_W, readout_b):
    src = edge_index[0]
    dst = edge_index[1]
    npad = _EPAD - _E
    zi = jnp.zeros((npad,), jnp.int32)
    dst_g = jnp.concatenate([dst, zi]).reshape(_IDXROWS, 128)
    src_g = jnp.concatenate([src, zi]).reshape(_IDXROWS, 128)
    dst_s = jnp.concatenate([dst, jnp.full((npad,), _N, jnp.int32)]
                            ).reshape(_IDXROWS, 128)
    zrows = jnp.zeros((_RPT, _SW), jnp.float32)

    x = pl.pallas_call(
        _readin_body,
        out_shape=jax.ShapeDtypeStruct((_N, 32), jnp.float32),
    )(h, readin_W, readin_b.reshape(1, -1))

    p = pos
    L = bn_gamma.shape[0]
    for l in range(L):
        w1a = edge_W1[l, :32]
        w1b = edge_W1[l, 32:64]
        w1c = edge_W1[l, 64:65]
        y, t1, t2 = pl.pallas_call(
            _prep_body,
            out_shape=[
                jax.ShapeDtypeStruct((_N, 32), jnp.float32),
                jax.ShapeDtypeStruct((_N, _GW), jnp.float32),
                jax.ShapeDtypeStruct((_N, _GW), jnp.float32),
            ],
        )(x, p, bn_gamma[l].reshape(1, -1), bn_beta[l].reshape(1, -1),
          w1a, w1b, edge_b1[l].reshape(1, -1))

        g = _sc_gather(t1, t2, dst_g, src_g)

        r = _edge_call(g, w1c, edge_W2[l], edge_b2[l].reshape(1, -1),
                       coord_W1[l], coord_b1[l].reshape(1, -1),
                       coord_W2[l], coord_b2[l].reshape(1, -1))

        parts = _sc_scatter(r, dst_s, zrows)

        x, p = pl.pallas_call(
            _node_body,
            out_shape=[
                jax.ShapeDtypeStruct((_N, 32), jnp.float32),
                jax.ShapeDtypeStruct((_N, 3), jnp.float32),
            ],
        )(x, p, y, parts, node_W1[l], node_b1[l].reshape(1, -1),
          node_W2[l], node_b2[l].reshape(1, -1))

    return pl.pallas_call(
        _readout_body,
        out_shape=jax.ShapeDtypeStruct((_N, 128), jnp.float32),
    )(x, readout_W, readout_b.reshape(1, -1))


# speedup vs baseline: 1.1955x; 1.0889x over previous
import functools

import jax
import jax.numpy as jnp
from jax import lax
from jax.experimental import pallas as pl
from jax.experimental.pallas import tpu as pltpu
from jax.experimental.pallas import tpu_sc as plsc

_N = 10000
_E = 320000
_EPAD = 327680
_IDXROWS = _EPAD // 128
_ROWS_PER_W = _IDXROWS // 32
_NCHUNK = 10
_GW = 48
_SW = 32
_NACC = _N + 16
_RPT = _NACC // 16


def _lrelu(x):
    return jnp.where(x > 0, x, 0.01 * x)


_mesh = plsc.VectorSubcoreMesh(core_axis_name="c", subcore_axis_name="s")
_sc_params = pltpu.CompilerParams(use_tc_tiling_on_sc=False)


_GCH = 1024


@functools.partial(
    pl.kernel,
    out_type=jax.ShapeDtypeStruct((_EPAD, _GW), jnp.float32),
    mesh=_mesh,
    compiler_params=_sc_params,
    scratch_types=[
        pltpu.VMEM((8, 128), jnp.int32),
        pltpu.VMEM((8, 128), jnp.int32),
        pltpu.VMEM((8, 128), jnp.int32),
        pltpu.VMEM((8, 128), jnp.int32),
        pltpu.VMEM((_GCH, _GW), jnp.float32),
        pltpu.VMEM((_GCH, _GW), jnp.float32),
        pltpu.SemaphoreType.DMA,
        pltpu.SemaphoreType.DMA,
        pltpu.SemaphoreType.DMA,
        pltpu.SemaphoreType.DMA,
    ],
)
def _sc_gather(t1, t2, dst2d, src2d, out,
               ida, idb, isa, isb, ba, bb, sga, sgb, swa, swb):
    c = lax.axis_index("c")
    s = lax.axis_index("s")
    wid = s * 2 + c
    base = wid * _ROWS_PER_W
    idd = [ida, idb]
    ids = [isa, isb]
    buf = [ba, bb]
    sg = [sga, sgb]
    sw = [swa, swb]

    def load_idx(ci, k):
        r0 = base + ci * 8
        pltpu.sync_copy(dst2d.at[pl.ds(r0, 8)], idd[k])
        pltpu.sync_copy(src2d.at[pl.ds(r0, 8)], ids[k])

    def fire_t1(k):
        return [pltpu.async_copy(t1.at[idd[k].at[j]],
                                 buf[k].at[pl.ds(j * 128, 128)], sg[k])
                for j in range(8)]

    def fire_t2(k):
        return [pltpu.async_copy(t2.at[ids[k].at[j]],
                                 buf[k].at[pl.ds(j * 128, 128)], sg[k],
                                 add=True)
                for j in range(8)]

    wo = [None, None]
    t1g = [None, None]
    load_idx(0, 0)
    t1g[0] = fire_t1(0)
    for ci in range(_NCHUNK):
        k = ci & 1
        kn = 1 - k
        if ci + 1 < _NCHUNK:
            load_idx(ci + 1, kn)
            if wo[kn] is not None:
                wo[kn].wait()
            t1g[kn] = fire_t1(kn)
        for cp in t1g[k]:
            cp.wait()
        for cp in fire_t2(k):
            cp.wait()
        wo[k] = pltpu.async_copy(
            buf[k], out.at[pl.ds((base + ci * 8) * 128, _GCH)], sw[k])
    wo[0].wait()
    wo[1].wait()


@functools.partial(
    pl.kernel,
    out_type=jax.ShapeDtypeStruct((2, _NACC, _SW), jnp.float32),
    mesh=_mesh,
    compiler_params=_sc_params,
    scratch_types=[
        pltpu.VMEM((8, 128), jnp.int32),
        pltpu.VMEM((1024, _SW), jnp.float32),
        pltpu.VMEM_SHARED((_NACC, _SW), jnp.float32),
    ],
)
def _sc_scatter(r_hbm, dst2d, zrows, out, idxv, rb, acc):
    c = lax.axis_index("c")
    s = lax.axis_index("s")
    wid = s * 2 + c
    base = wid * _ROWS_PER_W

    pltpu.sync_copy(zrows, rb.at[pl.ds(0, _RPT)])
    pltpu.sync_copy(rb.at[pl.ds(0, _RPT)], acc.at[pl.ds(s * _RPT, _RPT)])
    plsc.subcore_barrier()

    def chunk(ci, carry):
        r0 = base + ci * 8
        pltpu.sync_copy(dst2d.at[pl.ds(r0, 8)], idxv)
        pltpu.sync_copy(r_hbm.at[pl.ds(r0 * 128, 1024)], rb)
        for j in range(8):
            pltpu.sync_copy(rb.at[pl.ds(j * 128, 128)],
                            acc.at[idxv.at[j]], add=True)
        return carry

    lax.fori_loop(0, _NCHUNK, chunk, 0)
    plsc.subcore_barrier()
    pltpu.sync_copy(acc.at[pl.ds(s * _RPT, _RPT)], rb.at[pl.ds(0, _RPT)])
    pltpu.sync_copy(rb.at[pl.ds(0, _RPT)], out.at[c, pl.ds(s * _RPT, _RPT)])


def _prep0_body(h_ref, p_ref, rw_ref, rb_ref, g_ref, bt_ref, w1a_ref,
                w1b_ref, b1_ref, x_ref_o, y_ref, t1_ref, t2_ref):
    x = _lrelu(
        jnp.dot(h_ref[...], rw_ref[...], preferred_element_type=jnp.float32)
        + rb_ref[...])
    x_ref_o[...] = x
    _prep_common(x, p_ref, g_ref, bt_ref, w1a_ref, w1b_ref, b1_ref,
                 y_ref, t1_ref, t2_ref)


def _prep_body(x_ref, p_ref, g_ref, bt_ref, w1a_ref, w1b_ref, b1_ref,
               y_ref, t1_ref, t2_ref):
    _prep_common(x_ref[...], p_ref, g_ref, bt_ref, w1a_ref, w1b_ref, b1_ref,
                 y_ref, t1_ref, t2_ref)


def _prep_common(x, p_ref, g_ref, bt_ref, w1a_ref, w1b_ref, b1_ref,
                 y_ref, t1_ref, t2_ref):
    mu = jnp.mean(x, axis=0, keepdims=True)
    xc = x - mu
    var = jnp.mean(xc * xc, axis=0, keepdims=True)
    y = _lrelu(xc / jnp.sqrt(var + 1e-5) * g_ref[...] + bt_ref[...])
    y_ref[...] = y
    p = p_ref[...]
    zpad = jnp.zeros((_N, _GW - 35), jnp.float32)
    a = jnp.dot(y, w1a_ref[...], preferred_element_type=jnp.float32) + b1_ref[...]
    b = jnp.dot(y, w1b_ref[...], preferred_element_type=jnp.float32)
    t1_ref[...] = jnp.concatenate([a, p, zpad], axis=1)
    t2_ref[...] = jnp.concatenate([b, -p, zpad], axis=1)


_K = 8
_GP = _K * _GW
_MP = _K * 16
_UP = _K * 32
_XP = _K * 4
_RP = _K * _SW


def _edge_packed_weights(w1c, W2, b2, cW1, cb1, cW2, cb2):
    import numpy as np
    eyeK = np.eye(_K, dtype=np.float32)
    S1 = np.zeros((_GW, _GW), np.float32)
    S1[32:35, 0:32] = 1.0
    D1 = np.zeros((_GW, 4), np.float32)
    D1[32:35, 0:3] = np.eye(3)
    Pm1 = np.zeros((16, _SW), np.float32)
    Pm1[:, :16] = np.eye(16)
    Px1 = np.zeros((4, _SW), np.float32)
    Px1[0:3, 16:19] = np.eye(3)
    onesK = np.ones((_K,), np.float32)

    w2pad = jnp.pad(W2, ((0, _GW - 32), (0, 0)))
    cw2pad = jnp.pad(cW2, ((0, 0), (0, 1)))
    return dict(
        sp=jnp.asarray(np.kron(eyeK, S1)),
        dsel=jnp.asarray(np.kron(eyeK, D1)),
        pm=jnp.asarray(np.kron(eyeK, Pm1)),
        px=jnp.asarray(np.kron(eyeK, Px1)),
        w1cp=jnp.kron(onesK, jnp.pad(w1c.reshape(-1), (0, _GW - 32))).reshape(1, _GP),
        w2p=jnp.kron(jnp.asarray(eyeK), w2pad),
        b2p=jnp.kron(onesK, b2.reshape(-1)).reshape(1, _MP),
        cw1p=jnp.kron(jnp.asarray(eyeK), cW1),
        cb1p=jnp.kron(onesK, cb1.reshape(-1)).reshape(1, _UP),
        cw2p=jnp.kron(jnp.asarray(eyeK), cw2pad),
        cb2p=jnp.kron(onesK, jnp.pad(cb2.reshape(-1), (0, 1))).reshape(1, _XP),
    )


def _edge_body(g_ref, sp_ref, dsel_ref, pm_ref, px_ref, w1cp_ref, w2p_ref,
               b2p_ref, cw1p_ref, cb1p_ref, cw2p_ref, cb2p_ref, o_ref):
    g = g_ref[...]
    sq = g * g
    dist = jnp.sqrt(
        jnp.dot(sq, sp_ref[...], preferred_element_type=jnp.float32) + 1e-8)
    t = _lrelu(g + dist * w1cp_ref[...])
    m = jnp.dot(t, w2p_ref[...], preferred_element_type=jnp.float32) + b2p_ref[...]
    u = _lrelu(jnp.dot(m, cw1p_ref[...], preferred_element_type=jnp.float32)
               + cb1p_ref[...])
    cw = (jnp.dot(u, cw2p_ref[...], preferred_element_type=jnp.float32)
          + cb2p_ref[...])
    dsv = jnp.dot(g, dsel_ref[...], preferred_element_type=jnp.float32)
    xij = dsv * cw
    o_ref[...] = (jnp.dot(m, pm_ref[...], preferred_element_type=jnp.float32)
                  + jnp.dot(xij, px_ref[...], preferred_element_type=jnp.float32))


def _node_common(x_ref, y_ref, parts_ref, nw1_ref, nb1_ref, nw2_ref, nb2_ref):
    agg = parts_ref[0, :_N, :] + parts_ref[1, :_N, :]
    aggx = agg[:, 16:19]
    cat = jnp.concatenate([y_ref[...], agg[:, :16]], axis=1)
    t = _lrelu(jnp.dot(cat, nw1_ref[...], preferred_element_type=jnp.float32)
               + nb1_ref[...])
    hn = jnp.dot(t, nw2_ref[...], preferred_element_type=jnp.float32) + nb2_ref[...]
    return x_ref[...] + hn, aggx


def _node_body(x_ref, p_ref, y_ref, parts_ref, nw1_ref, nb1_ref, nw2_ref,
               nb2_ref, xo_ref, po_ref):
    xn, aggx = _node_common(x_ref, y_ref, parts_ref, nw1_ref, nb1_ref,
                            nw2_ref, nb2_ref)
    xo_ref[...] = xn
    po_ref[...] = p_ref[...] + aggx


def _node_out_body(x_ref, y_ref, parts_ref, nw1_ref, nb1_ref, nw2_ref,
                   nb2_ref, ow_ref, ob_ref, o_ref):
    xn, _ = _node_common(x_ref, y_ref, parts_ref, nw1_ref, nb1_ref,
                         nw2_ref, nb2_ref)
    o_ref[...] = (jnp.dot(xn, ow_ref[...], preferred_element_type=jnp.float32)
                  + ob_ref[...])


_RB = 1024


def _edge_call(g, w1c, w2, b2, cw1, cb1, cw2, cb2):
    wd = _edge_packed_weights(w1c, w2, b2, cw1, cb1, cw2, cb2)
    gp = g.reshape(_EPAD // _K, _GP)
    nblk = gp.shape[0] // _RB
    full = lambda a: pl.BlockSpec(a.shape, lambda i: tuple(0 for _ in a.shape))
    args = [wd["sp"], wd["dsel"], wd["pm"], wd["px"], wd["w1cp"], wd["w2p"],
            wd["b2p"], wd["cw1p"], wd["cb1p"], wd["cw2p"], wd["cb2p"]]
    rp = pl.pallas_call(
        _edge_body,
        grid=(nblk,),
        in_specs=[pl.BlockSpec((_RB, _GP), lambda i: (i, 0))] +
                 [full(a) for a in args],
        out_specs=pl.BlockSpec((_RB, _RP), lambda i: (i, 0)),
        out_shape=jax.ShapeDtypeStruct((_EPAD // _K, _RP), jnp.float32),
    )(gp, *args)
    return rp.reshape(_EPAD, _SW)


def kernel(h, pos, edge_index, readin_W, readin_b, bn_gamma, bn_beta,
           edge_W1, edge_b1, edge_W2, edge_b2,
           coord_W1, coord_b1, coord_W2, coord_b2,
           node_W1, node_b1, node_W2, node_b2,
           readout_W, readout_b):
    src = edge_index[0]
    dst = edge_index[1]
    npad = _EPAD - _E
    zi = jnp.zeros((npad,), jnp.int32)
    dst_g = jnp.concatenate([dst, zi]).reshape(_IDXROWS, 128)
    src_g = jnp.concatenate([src, zi]).reshape(_IDXROWS, 128)
    dst_s = jnp.concatenate([dst, jnp.full((npad,), _N, jnp.int32)]
                            ).reshape(_IDXROWS, 128)
    zrows = jnp.zeros((_RPT, _SW), jnp.float32)

    x = None
    p = pos
    L = bn_gamma.shape[0]
    out = None
    for l in range(L):
        w1a = edge_W1[l, :32]
        w1b = edge_W1[l, 32:64]
        w1c = edge_W1[l, 64:65]
        prep_shapes = [
            jax.ShapeDtypeStruct((_N, 32), jnp.float32),
            jax.ShapeDtypeStruct((_N, _GW), jnp.float32),
            jax.ShapeDtypeStruct((_N, _GW), jnp.float32),
        ]
        bnw = (bn_gamma[l].reshape(1, -1), bn_beta[l].reshape(1, -1),
               w1a, w1b, edge_b1[l].reshape(1, -1))
        if l == 0:
            x, y, t1, t2 = pl.pallas_call(
                _prep0_body,
                out_shape=[jax.ShapeDtypeStruct((_N, 32), jnp.float32)]
                + prep_shapes,
            )(h, p, readin_W, readin_b.reshape(1, -1), *bnw)
        else:
            y, t1, t2 = pl.pallas_call(
                _prep_body, out_shape=prep_shapes)(x, p, *bnw)

        g = _sc_gather(t1, t2, dst_g, src_g)

        r = _edge_call(g, w1c, edge_W2[l], edge_b2[l].reshape(1, -1),
                       coord_W1[l], coord_b1[l].reshape(1, -1),
                       coord_W2[l], coord_b2[l].reshape(1, -1))

        parts = _sc_scatter(r, dst_s, zrows)

        nodew = (node_W1[l], node_b1[l].reshape(1, -1),
                 node_W2[l], node_b2[l].reshape(1, -1))
        if l == L - 1:
            out = pl.pallas_call(
                _node_out_body,
                out_shape=jax.ShapeDtypeStruct((_N, 128), jnp.float32),
            )(x, y, parts, *nodew, readout_W, readout_b.reshape(1, -1))
        else:
            x, p = pl.pallas_call(
                _node_body,
                out_shape=[
                    jax.ShapeDtypeStruct((_N, 32), jnp.float32),
                    jax.ShapeDtypeStruct((_N, 3), jnp.float32),
                ],
            )(x, p, y, parts, *nodew)

    return out


# speedup vs baseline: 1.1996x; 1.0034x over previous
import functools

import jax
import jax.numpy as jnp
from jax import lax
from jax.experimental import pallas as pl
from jax.experimental.pallas import tpu as pltpu
from jax.experimental.pallas import tpu_sc as plsc

_N = 10000
_E = 320000
_EPAD = 327680
_IDXROWS = _EPAD // 128
_ROWS_PER_W = _IDXROWS // 32
_NCHUNK = 10
_GW = 48
_SW = 32
_NACC = _N + 16
_RPT = _NACC // 16


def _lrelu(x):
    return jnp.where(x > 0, x, 0.01 * x)


_mesh = plsc.VectorSubcoreMesh(core_axis_name="c", subcore_axis_name="s")
_sc_params = pltpu.CompilerParams(use_tc_tiling_on_sc=False)


_GCH = 1024


@functools.partial(
    pl.kernel,
    out_type=jax.ShapeDtypeStruct((_EPAD, _GW), jnp.float32),
    mesh=_mesh,
    compiler_params=_sc_params,
    scratch_types=[
        pltpu.VMEM((8, 128), jnp.int32),
        pltpu.VMEM((8, 128), jnp.int32),
        pltpu.VMEM((8, 128), jnp.int32),
        pltpu.VMEM((8, 128), jnp.int32),
        pltpu.VMEM((_GCH, _GW), jnp.float32),
        pltpu.VMEM((_GCH, _GW), jnp.float32),
        pltpu.SemaphoreType.DMA,
        pltpu.SemaphoreType.DMA,
        pltpu.SemaphoreType.DMA,
        pltpu.SemaphoreType.DMA,
    ],
)
def _sc_gather(t1, t2, dst2d, src2d, out,
               ida, idb, isa, isb, ba, bb, sga, sgb, swa, swb):
    c = lax.axis_index("c")
    s = lax.axis_index("s")
    wid = c * 16 + s
    base = wid * _ROWS_PER_W
    idd = [ida, idb]
    ids = [isa, isb]
    buf = [ba, bb]
    sg = [sga, sgb]
    sw = [swa, swb]

    def load_idx(ci, k):
        r0 = base + ci * 8
        pltpu.sync_copy(dst2d.at[pl.ds(r0, 8)], idd[k])
        pltpu.sync_copy(src2d.at[pl.ds(r0, 8)], ids[k])

    def fire_t1(k):
        return [pltpu.async_copy(t1.at[idd[k].at[j]],
                                 buf[k].at[pl.ds(j * 128, 128)], sg[k])
                for j in range(8)]

    def fire_t2(k):
        return [pltpu.async_copy(t2.at[ids[k].at[j]],
                                 buf[k].at[pl.ds(j * 128, 128)], sg[k],
                                 add=True)
                for j in range(8)]

    wo = [None, None]
    t1g = [None, None]
    load_idx(0, 0)
    t1g[0] = fire_t1(0)
    for ci in range(_NCHUNK):
        k = ci & 1
        kn = 1 - k
        if ci + 1 < _NCHUNK:
            load_idx(ci + 1, kn)
            if wo[kn] is not None:
                wo[kn].wait()
            t1g[kn] = fire_t1(kn)
        for cp in t1g[k]:
            cp.wait()
        for cp in fire_t2(k):
            cp.wait()
        wo[k] = pltpu.async_copy(
            buf[k], out.at[pl.ds((base + ci * 8) * 128, _GCH)], sw[k])
    wo[0].wait()
    wo[1].wait()


@functools.partial(
    pl.kernel,
    out_type=jax.ShapeDtypeStruct((2, _NACC, _SW), jnp.float32),
    mesh=_mesh,
    compiler_params=_sc_params,
    scratch_types=[
        pltpu.VMEM((8, 128), jnp.int32),
        pltpu.VMEM((1024, _SW), jnp.float32),
        pltpu.VMEM_SHARED((_NACC, _SW), jnp.float32),
    ],
)
def _sc_scatter(r_hbm, dst2d, zrows, out, idxv, rb, acc):
    c = lax.axis_index("c")
    s = lax.axis_index("s")
    wid = c * 16 + s
    base = wid * _ROWS_PER_W

    pltpu.sync_copy(zrows, rb.at[pl.ds(0, _RPT)])
    pltpu.sync_copy(rb.at[pl.ds(0, _RPT)], acc.at[pl.ds(s * _RPT, _RPT)])
    plsc.subcore_barrier()

    def chunk(ci, carry):
        r0 = base + ci * 8
        pltpu.sync_copy(dst2d.at[pl.ds(r0, 8)], idxv)
        pltpu.sync_copy(r_hbm.at[pl.ds(r0 * 128, 1024)], rb)
        for j in range(8):
            pltpu.sync_copy(rb.at[pl.ds(j * 128, 128)],
                            acc.at[idxv.at[j]], add=True)
        return carry

    lax.fori_loop(0, _NCHUNK, chunk, 0)
    plsc.subcore_barrier()
    pltpu.sync_copy(acc.at[pl.ds(s * _RPT, _RPT)], rb.at[pl.ds(0, _RPT)])
    pltpu.sync_copy(rb.at[pl.ds(0, _RPT)], out.at[c, pl.ds(s * _RPT, _RPT)])


def _prep0_body(h_ref, p_ref, rw_ref, rb_ref, g_ref, bt_ref, w1a_ref,
                w1b_ref, b1_ref, x_ref_o, y_ref, t1_ref, t2_ref):
    x = _lrelu(
        jnp.dot(h_ref[...], rw_ref[...], preferred_element_type=jnp.float32)
        + rb_ref[...])
    x_ref_o[...] = x
    _prep_common(x, p_ref, g_ref, bt_ref, w1a_ref, w1b_ref, b1_ref,
                 y_ref, t1_ref, t2_ref)


def _prep_body(x_ref, p_ref, g_ref, bt_ref, w1a_ref, w1b_ref, b1_ref,
               y_ref, t1_ref, t2_ref):
    _prep_common(x_ref[...], p_ref, g_ref, bt_ref, w1a_ref, w1b_ref, b1_ref,
                 y_ref, t1_ref, t2_ref)


def _prep_common(x, p_ref, g_ref, bt_ref, w1a_ref, w1b_ref, b1_ref,
                 y_ref, t1_ref, t2_ref):
    mu = jnp.mean(x, axis=0, keepdims=True)
    xc = x - mu
    var = jnp.mean(xc * xc, axis=0, keepdims=True)
    y = _lrelu(xc / jnp.sqrt(var + 1e-5) * g_ref[...] + bt_ref[...])
    y_ref[...] = y
    p = p_ref[...]
    zpad = jnp.zeros((_N, _GW - 35), jnp.float32)
    a = jnp.dot(y, w1a_ref[...], preferred_element_type=jnp.float32) + b1_ref[...]
    b = jnp.dot(y, w1b_ref[...], preferred_element_type=jnp.float32)
    t1_ref[...] = jnp.concatenate([a, p, zpad], axis=1)
    t2_ref[...] = jnp.concatenate([b, -p, zpad], axis=1)


_K = 8
_GP = _K * _GW
_MP = _K * 16
_UP = _K * 32
_XP = _K * 4
_RP = _K * _SW


def _edge_packed_weights(w1c, W2, b2, cW1, cb1, cW2, cb2):
    import numpy as np
    eyeK = np.eye(_K, dtype=np.float32)
    S1 = np.zeros((_GW, _GW), np.float32)
    S1[32:35, 0:32] = 1.0
    D1 = np.zeros((_GW, 4), np.float32)
    D1[32:35, 0:3] = np.eye(3)
    Pm1 = np.zeros((16, _SW), np.float32)
    Pm1[:, :16] = np.eye(16)
    Px1 = np.zeros((4, _SW), np.float32)
    Px1[0:3, 16:19] = np.eye(3)
    onesK = np.ones((_K,), np.float32)

    w2pad = jnp.pad(W2, ((0, _GW - 32), (0, 0)))
    cw2pad = jnp.pad(cW2, ((0, 0), (0, 1)))
    return dict(
        sp=jnp.asarray(np.kron(eyeK, S1)),
        dsel=jnp.asarray(np.kron(eyeK, D1)),
        pm=jnp.asarray(np.kron(eyeK, Pm1)),
        px=jnp.asarray(np.kron(eyeK, Px1)),
        w1cp=jnp.kron(onesK, jnp.pad(w1c.reshape(-1), (0, _GW - 32))).reshape(1, _GP),
        w2p=jnp.kron(jnp.asarray(eyeK), w2pad),
        b2p=jnp.kron(onesK, b2.reshape(-1)).reshape(1, _MP),
        cw1p=jnp.kron(jnp.asarray(eyeK), cW1),
        cb1p=jnp.kron(onesK, cb1.reshape(-1)).reshape(1, _UP),
        cw2p=jnp.kron(jnp.asarray(eyeK), cw2pad),
        cb2p=jnp.kron(onesK, jnp.pad(cb2.reshape(-1), (0, 1))).reshape(1, _XP),
    )


def _edge_body(g_ref, sp_ref, dsel_ref, pm_ref, px_ref, w1cp_ref, w2p_ref,
               b2p_ref, cw1p_ref, cb1p_ref, cw2p_ref, cb2p_ref, o_ref):
    g = g_ref[...]
    sq = g * g
    dist = jnp.sqrt(
        jnp.dot(sq, sp_ref[...], preferred_element_type=jnp.float32) + 1e-8)
    t = _lrelu(g + dist * w1cp_ref[...])
    m = jnp.dot(t, w2p_ref[...], preferred_element_type=jnp.float32) + b2p_ref[...]
    u = _lrelu(jnp.dot(m, cw1p_ref[...], preferred_element_type=jnp.float32)
               + cb1p_ref[...])
    cw = (jnp.dot(u, cw2p_ref[...], preferred_element_type=jnp.float32)
          + cb2p_ref[...])
    dsv = jnp.dot(g, dsel_ref[...], preferred_element_type=jnp.float32)
    xij = dsv * cw
    o_ref[...] = (jnp.dot(m, pm_ref[...], preferred_element_type=jnp.float32)
                  + jnp.dot(xij, px_ref[...], preferred_element_type=jnp.float32))


def _node_common(x_ref, y_ref, parts_ref, nw1_ref, nb1_ref, nw2_ref, nb2_ref):
    agg = parts_ref[0, :_N, :] + parts_ref[1, :_N, :]
    aggx = agg[:, 16:19]
    cat = jnp.concatenate([y_ref[...], agg[:, :16]], axis=1)
    t = _lrelu(jnp.dot(cat, nw1_ref[...], preferred_element_type=jnp.float32)
               + nb1_ref[...])
    hn = jnp.dot(t, nw2_ref[...], preferred_element_type=jnp.float32) + nb2_ref[...]
    return x_ref[...] + hn, aggx


def _node_body(x_ref, p_ref, y_ref, parts_ref, nw1_ref, nb1_ref, nw2_ref,
               nb2_ref, xo_ref, po_ref):
    xn, aggx = _node_common(x_ref, y_ref, parts_ref, nw1_ref, nb1_ref,
                            nw2_ref, nb2_ref)
    xo_ref[...] = xn
    po_ref[...] = p_ref[...] + aggx


def _node_out_body(x_ref, y_ref, parts_ref, nw1_ref, nb1_ref, nw2_ref,
                   nb2_ref, ow_ref, ob_ref, o_ref):
    xn, _ = _node_common(x_ref, y_ref, parts_ref, nw1_ref, nb1_ref,
                         nw2_ref, nb2_ref)
    o_ref[...] = (jnp.dot(xn, ow_ref[...], preferred_element_type=jnp.float32)
                  + ob_ref[...])


_RB = 1024


def _edge_call(g, w1c, w2, b2, cw1, cb1, cw2, cb2):
    wd = _edge_packed_weights(w1c, w2, b2, cw1, cb1, cw2, cb2)
    gp = g.reshape(_EPAD // _K, _GP)
    nblk = gp.shape[0] // _RB
    full = lambda a: pl.BlockSpec(a.shape, lambda i: tuple(0 for _ in a.shape))
    args = [wd["sp"], wd["dsel"], wd["pm"], wd["px"], wd["w1cp"], wd["w2p"],
            wd["b2p"], wd["cw1p"], wd["cb1p"], wd["cw2p"], wd["cb2p"]]
    rp = pl.pallas_call(
        _edge_body,
        grid=(nblk,),
        in_specs=[pl.BlockSpec((_RB, _GP), lambda i: (i, 0))] +
                 [full(a) for a in args],
        out_specs=pl.BlockSpec((_RB, _RP), lambda i: (i, 0)),
        out_shape=jax.ShapeDtypeStruct((_EPAD // _K, _RP), jnp.float32),
    )(gp, *args)
    return rp.reshape(_EPAD, _SW)


def kernel(h, pos, edge_index, readin_W, readin_b, bn_gamma, bn_beta,
           edge_W1, edge_b1, edge_W2, edge_b2,
           coord_W1, coord_b1, coord_W2, coord_b2,
           node_W1, node_b1, node_W2, node_b2,
           readout_W, readout_b):
    src = edge_index[0]
    dst = edge_index[1]
    npad = _EPAD - _E
    zi = jnp.zeros((npad,), jnp.int32)
    dst_g = jnp.concatenate([dst, zi]).reshape(_IDXROWS, 128)
    src_g = jnp.concatenate([src, zi]).reshape(_IDXROWS, 128)
    dst_s = jnp.concatenate([dst, jnp.full((npad,), _N, jnp.int32)]
                            ).reshape(_IDXROWS, 128)
    zrows = jnp.zeros((_RPT, _SW), jnp.float32)

    x = None
    p = pos
    L = bn_gamma.shape[0]
    out = None
    for l in range(L):
        w1a = edge_W1[l, :32]
        w1b = edge_W1[l, 32:64]
        w1c = edge_W1[l, 64:65]
        prep_shapes = [
            jax.ShapeDtypeStruct((_N, 32), jnp.float32),
            jax.ShapeDtypeStruct((_N, _GW), jnp.float32),
            jax.ShapeDtypeStruct((_N, _GW), jnp.float32),
        ]
        bnw = (bn_gamma[l].reshape(1, -1), bn_beta[l].reshape(1, -1),
               w1a, w1b, edge_b1[l].reshape(1, -1))
        if l == 0:
            x, y, t1, t2 = pl.pallas_call(
                _prep0_body,
                out_shape=[jax.ShapeDtypeStruct((_N, 32), jnp.float32)]
                + prep_shapes,
            )(h, p, readin_W, readin_b.reshape(1, -1), *bnw)
        else:
            y, t1, t2 = pl.pallas_call(
                _prep_body, out_shape=prep_shapes)(x, p, *bnw)

        g = _sc_gather(t1, t2, dst_g, src_g)

        r = _edge_call(g, w1c, edge_W2[l], edge_b2[l].reshape(1, -1),
                       coord_W1[l], coord_b1[l].reshape(1, -1),
                       coord_W2[l], coord_b2[l].reshape(1, -1))

        parts = _sc_scatter(r, dst_s, zrows)

        nodew = (node_W1[l], node_b1[l].reshape(1, -1),
                 node_W2[l], node_b2[l].reshape(1, -1))
        if l == L - 1:
            out = pl.pallas_call(
                _node_out_body,
                out_shape=jax.ShapeDtypeStruct((_N, 128), jnp.float32),
            )(x, y, parts, *nodew, readout_W, readout_b.reshape(1, -1))
        else:
            x, p = pl.pallas_call(
                _node_body,
                out_shape=[
                    jax.ShapeDtypeStruct((_N, 32), jnp.float32),
                    jax.ShapeDtypeStruct((_N, 3), jnp.float32),
                ],
            )(x, p, y, parts, *nodew)

    return out
